# baseline (speedup 1.0000x reference)
import functools

import jax
import jax.numpy as jnp
from jax import lax
from jax.experimental import pallas as pl
from jax.experimental.pallas import tpu as pltpu
from jax.experimental.pallas import tpu_sc as plsc

N = 10000
E = 640000
IN = 84
HID = 512
ED = 6
NGF = 16
NG = 128

FC = 128
NCH = HID // FC
EB = 128
NW = 32
EPW0 = E // NW
NB = 160
EPW = NB * EB
PAD = EPW - EPW0
IG = 16
NGRP = NB // IG
NP = 10240
RPT = NP // 16
ZR = 128
RB = 1000
G = N // RB


def _sc_body(*refs):
    (sd_h, zer_h, *rest) = refs
    tables = rest[:NCH]
    aouts = rest[NCH:2 * NCH]
    (sda, sdb, bufa, bufb, acc, sema, semb, sesa, sesb) = rest[2 * NCH:]

    c = lax.axis_index("c")
    s = lax.axis_index("s")
    w = c * 16 + s
    row0 = s * RPT

    for ch in range(NCH):
        tbl = tables[ch]

        pltpu.sync_copy(zer_h, acc.at[pl.ds(row0, RPT)])
        plsc.subcore_barrier()

        pltpu.sync_copy(sd_h.at[w, 0], sda)
        pltpu.async_copy(tbl.at[sda.at[0]], bufa, sema)

        @pl.loop(0, NB, step=2)
        def _edges(j):
            pltpu.sync_copy(sd_h.at[w, j + 1], sdb)
            pltpu.make_async_copy(tbl.at[sda.at[0]], bufa, sema).wait()
            pltpu.async_copy(tbl.at[sdb.at[0]], bufb, semb)
            pltpu.async_copy(bufa, acc.at[sda.at[1]], sesa, add=True)
            pltpu.make_async_copy(tbl.at[sdb.at[0]], bufb, semb).wait()
            pltpu.async_copy(bufb, acc.at[sdb.at[1]], sesb, add=True)
            pltpu.make_async_copy(bufa, acc.at[sda.at[1]], sesa).wait()
            pltpu.sync_copy(sd_h.at[w, lax.rem(j + 2, NB)], sda)
            pltpu.async_copy(tbl.at[sda.at[0]], bufa, sema)
            pltpu.make_async_copy(bufb, acc.at[sdb.at[1]], sesb).wait()

        pltpu.make_async_copy(tbl.at[sda.at[0]], bufa, sema).wait()

        plsc.subcore_barrier()
        pltpu.sync_copy(acc.at[pl.ds(row0, RPT)], aouts[ch].at[c, pl.ds(row0, RPT)])


_sc_agg = pl.kernel(
    _sc_body,
    out_type=[jax.ShapeDtypeStruct((2, NP, FC), jnp.float32) for _ in range(NCH)],
    mesh=plsc.VectorSubcoreMesh(core_axis_name="c", subcore_axis_name="s"),
    scratch_types=[
        pltpu.VMEM((2, EB), jnp.int32),
        pltpu.VMEM((2, EB), jnp.int32),
        pltpu.VMEM((EB, FC), jnp.float32),
        pltpu.VMEM((EB, FC), jnp.float32),
        pltpu.VMEM_SHARED((NP, FC), jnp.float32),
        pltpu.SemaphoreType.DMA,
        pltpu.SemaphoreType.DMA,
        pltpu.SemaphoreType.DMA,
        pltpu.SemaphoreType.DMA,
    ],
    name="sc_edge_agg",
)


def _sc_stats_body(dst_h, ea_h, zer_h, st_h, dst_i, erow, acc2):
    c = lax.axis_index("c")
    s = lax.axis_index("s")
    w = c * 16 + s
    row0 = s * RPT

    pltpu.sync_copy(zer_h, acc2.at[pl.ds(row0, RPT)])
    plsc.subcore_barrier()

    @pl.loop(0, NB)
    def _stats(j):
        pltpu.sync_copy(dst_h.at[pl.ds(w * EPW + j * EB, EB)], dst_i)
        pltpu.sync_copy(ea_h.at[pl.ds(w * EPW + j * EB, EB)], erow)
        pltpu.sync_copy(erow, acc2.at[dst_i], add=True)

    plsc.subcore_barrier()
    pltpu.sync_copy(acc2.at[pl.ds(row0, RPT)], st_h.at[c, pl.ds(row0, RPT)])


_sc_stats = pl.kernel(
    _sc_stats_body,
    out_type=[jax.ShapeDtypeStruct((2, NP, FC), jnp.float32)],
    mesh=plsc.VectorSubcoreMesh(core_axis_name="c", subcore_axis_name="s"),
    scratch_types=[
        pltpu.VMEM((EB,), jnp.int32),
        pltpu.VMEM((EB, FC), jnp.float32),
        pltpu.VMEM_SHARED((NP, FC), jnp.float32),
    ],
    name="sc_edge_stats",
)


def _k1_body(x_ref, wm_ref, ws_ref, bs_ref, *outs):
    xa = x_ref[...]
    pre = jnp.dot(xa, wm_ref[...], preferred_element_type=jnp.float32)
    for ch in range(NCH):
        outs[ch][...] = pre[:, ch * FC:(ch + 1) * FC]
    outs[NCH][...] = jnp.dot(xa, ws_ref[...],
                             preferred_element_type=jnp.float32) + bs_ref[...]


_k1 = pl.pallas_call(
    _k1_body,
    grid=(G,),
    in_specs=[
        pl.BlockSpec((RB, IN), lambda i: (i, 0)),
        pl.BlockSpec((IN, HID), lambda i: (0, 0)),
        pl.BlockSpec((IN, HID), lambda i: (0, 0)),
        pl.BlockSpec((1, HID), lambda i: (0, 0)),
    ],
    out_specs=[pl.BlockSpec((RB, FC), lambda i: (i, 0)) for _ in range(NCH)]
    + [pl.BlockSpec((RB, HID), lambda i: (i, 0))],
    out_shape=[jax.ShapeDtypeStruct((N, FC), jnp.float32) for _ in range(NCH)]
    + [jax.ShapeDtypeStruct((N, HID), jnp.float32)],
)


def _mid_body(residual, *refs):
    a_refs = refs[:NCH]
    st_ref, prev_ref, we_ref, bb_ref, wmn_ref, x_ref = refs[NCH:NCH + 6]
    q_refs = refs[NCH + 6:]
    aggs = []
    for a in a_refs:
        av = a[...]
        aggs.append(av[0] + av[1])
    agg = jnp.concatenate(aggs, axis=1)
    st = st_ref[...]
    sv = st[0] + st[1]
    d = jnp.dot(sv[:, :ED], we_ref[...], preferred_element_type=jnp.float32)
    d = d + sv[:, ED:ED + 1] * bb_ref[...]
    prev = prev_ref[...]
    xl = jnp.maximum(agg + d + prev, 0.0)
    if residual:
        xl = xl + prev
    x_ref[...] = xl
    q = jnp.dot(xl, wmn_ref[...], preferred_element_type=jnp.float32)
    for ch, qr in enumerate(q_refs):
        qr[...] = q[:, ch * FC:(ch + 1) * FC]


def _make_mid(residual):
    return pl.pallas_call(
        functools.partial(_mid_body, residual),
        grid=(G,),
        in_specs=[pl.BlockSpec((2, RB, FC), lambda i: (0, i, 0)) for _ in range(NCH)]
        + [
            pl.BlockSpec((2, RB, FC), lambda i: (0, i, 0)),
            pl.BlockSpec((RB, HID), lambda i: (i, 0)),
            pl.BlockSpec((ED, HID), lambda i: (0, 0)),
            pl.BlockSpec((1, HID), lambda i: (0, 0)),
            pl.BlockSpec((HID, HID), lambda i: (0, 0)),
        ],
        out_specs=[pl.BlockSpec((RB, HID), lambda i: (i, 0))]
        + [pl.BlockSpec((RB, FC), lambda i: (i, 0)) for _ in range(NCH)],
        out_shape=[jax.ShapeDtypeStruct((N, HID), jnp.float32)]
        + [jax.ShapeDtypeStruct((N, FC), jnp.float32) for _ in range(NCH)],
    )


_mid1 = _make_mid(False)
_mid2 = _make_mid(True)


def _e3_body(*refs):
    a_refs = refs[:NCH]
    st_ref, prev_ref, we_ref, bb_ref, b_ref, psum_ref, cnt_ref = refs[NCH:]
    i = pl.program_id(0)
    aggs = []
    for a in a_refs:
        av = a[...]
        aggs.append(av[0] + av[1])
    agg = jnp.concatenate(aggs, axis=1)
    st = st_ref[...]
    sv = st[0] + st[1]
    d = jnp.dot(sv[:, :ED], we_ref[...], preferred_element_type=jnp.float32)
    d = d + sv[:, ED:ED + 1] * bb_ref[...]
    prev = prev_ref[...]
    x3 = jnp.maximum(agg + d + prev, 0.0) + prev
    b = b_ref[0, 0]
    gid = lax.broadcasted_iota(jnp.int32, (NG, RB), 0)
    mask = (gid == b[None, :]).astype(jnp.float32)

    @pl.when(i == 0)
    def _():
        psum_ref[...] = jnp.zeros_like(psum_ref)
        cnt_ref[...] = jnp.zeros_like(cnt_ref)

    psum_ref[...] += jnp.dot(mask, x3, preferred_element_type=jnp.float32)
    cnt_ref[...] += jnp.sum(mask, axis=1, keepdims=True)


_e3 = pl.pallas_call(
    _e3_body,
    grid=(G,),
    in_specs=[pl.BlockSpec((2, RB, FC), lambda i: (0, i, 0)) for _ in range(NCH)]
    + [
        pl.BlockSpec((2, RB, FC), lambda i: (0, i, 0)),
        pl.BlockSpec((RB, HID), lambda i: (i, 0)),
        pl.BlockSpec((ED, HID), lambda i: (0, 0)),
        pl.BlockSpec((1, HID), lambda i: (0, 0)),
        pl.BlockSpec((1, 1, RB), lambda i: (i, 0, 0)),
    ],
    out_specs=[
        pl.BlockSpec((NG, HID), lambda i: (0, 0)),
        pl.BlockSpec((NG, 1), lambda i: (0, 0)),
    ],
    out_shape=[
        jax.ShapeDtypeStruct((NG, HID), jnp.float32),
        jax.ShapeDtypeStruct((NG, 1), jnp.float32),
    ],
)


def _fin_body(psum_ref, cnt_ref, ga_ref, wg_ref, bg_ref, wc_ref, bc_ref,
              wl_ref, bl_ref, out_ref):
    pooled = psum_ref[...] / jnp.maximum(cnt_ref[...], 1.0)
    g = jnp.dot(ga_ref[...], wg_ref[...],
                preferred_element_type=jnp.float32) + bg_ref[...]
    comb = jnp.concatenate([pooled, g], axis=1)
    h = jnp.maximum(jnp.dot(comb, wc_ref[...],
                            preferred_element_type=jnp.float32) + bc_ref[...], 0.0)
    out_ref[...] = jnp.dot(h, wl_ref[...],
                           preferred_element_type=jnp.float32) + bl_ref[...]


_fin = pl.pallas_call(
    _fin_body,
    out_shape=jax.ShapeDtypeStruct((NG, 1), jnp.float32),
)


def kernel(x, edge_index, edge_attr, batch, graph_attr,
           Wm1, bm1, We1, be1, Wm2, bm2, We2, be2,
           Wm3, bm3, We3, be3, Wm4, bm4, We4, be4,
           Ws1, bs1, Wg, bg, Wc, bc, Wl, bl):
    pad_dst = jnp.broadcast_to(
        N + (jnp.arange(PAD, dtype=jnp.int32) % (NP - N - 16)), (NW, PAD))
    src_w = jnp.concatenate(
        [edge_index[0].reshape(NW, EPW0),
         jnp.zeros((NW, PAD), jnp.int32)], axis=1).reshape(NW, NB, EB)
    dst_w = jnp.concatenate(
        [edge_index[1].reshape(NW, EPW0), pad_dst], axis=1).reshape(NW, NB, EB)
    sd = jnp.stack([src_w, dst_w], axis=2)
    dst = dst_w.reshape(NW * EPW)
    ea7 = jnp.concatenate([edge_attr, jnp.ones((E, 1), jnp.float32)], axis=1)
    ea7 = jnp.concatenate(
        [ea7.reshape(NW, EPW0, ED + 1),
         jnp.zeros((NW, PAD, ED + 1), jnp.float32)], axis=1)
    ea128 = jnp.pad(ea7.reshape(NW * EPW, ED + 1),
                    ((0, 0), (0, FC - ED - 1)))

    *p, xself = _k1(x, Wm1, Ws1, bs1.reshape(1, HID))

    zer = jnp.zeros((RPT, FC), jnp.float32)
    (stats,) = _sc_stats(dst, ea128, zer)
    a = _sc_agg(sd, zer, *p)
    x1, *q = _mid1(*a, stats, xself, We1, (bm1 + be1).reshape(1, HID), Wm2)

    a = _sc_agg(sd, zer, *q)
    x2, *q = _mid2(*a, stats, x1, We2, (bm2 + be2).reshape(1, HID), Wm3)

    a = _sc_agg(sd, zer, *q)
    psum, cnt = _e3(*a, stats, x2, We3,
                    (bm3 + be3).reshape(1, HID), batch.reshape(G, 1, RB))

    return _fin(psum, cnt, graph_attr, Wg, bg.reshape(1, HID),
                Wc, bc.reshape(1, HID), Wl, bl.reshape(1, 1))


# speedup vs baseline: 1.0288x; 1.0288x over previous
import functools

import jax
import jax.numpy as jnp
from jax import lax
from jax.experimental import pallas as pl
from jax.experimental.pallas import tpu as pltpu
from jax.experimental.pallas import tpu_sc as plsc

N = 10000
E = 640000
IN = 84
HID = 512
ED = 6
NGF = 16
NG = 128

FC = 128
NCH = HID // FC
EB = 128
NW = 32
EPW0 = E // NW
NB = 160
EPW = NB * EB
PAD = EPW - EPW0
IG = 16
NGRP = NB // IG
NP = 10240
RPT = NP // 16
ZR = 128
RB = 1000
G = N // RB


def _sc_body(*refs):
    (sd_h, zer_h, *rest) = refs
    tables = rest[:NCH]
    aouts = rest[NCH:2 * NCH]
    (sda, sdb, bufa, bufb, acc, sema, semb, sesa, sesb) = rest[2 * NCH:]

    c = lax.axis_index("c")
    s = lax.axis_index("s")
    w = c * 16 + s
    row0 = s * RPT

    for ch in range(NCH):
        tbl = tables[ch]

        pltpu.sync_copy(zer_h, acc.at[pl.ds(row0, RPT)])
        plsc.subcore_barrier()

        pltpu.sync_copy(sd_h.at[w, 0], sda)
        pltpu.async_copy(tbl.at[sda.at[0]], bufa, sema)

        @pl.loop(0, NB, step=2)
        def _edges(j):
            pltpu.sync_copy(sd_h.at[w, j + 1], sdb)
            pltpu.make_async_copy(tbl.at[sda.at[0]], bufa, sema).wait()
            pltpu.async_copy(tbl.at[sdb.at[0]], bufb, semb)
            pltpu.sync_copy(bufa, acc.at[sda.at[1]], add=True)
            pltpu.sync_copy(sd_h.at[w, lax.rem(j + 2, NB)], sda)
            pltpu.make_async_copy(tbl.at[sdb.at[0]], bufb, semb).wait()
            pltpu.async_copy(tbl.at[sda.at[0]], bufa, sema)
            pltpu.sync_copy(bufb, acc.at[sdb.at[1]], add=True)

        pltpu.make_async_copy(tbl.at[sda.at[0]], bufa, sema).wait()

        plsc.subcore_barrier()
        pltpu.sync_copy(acc.at[pl.ds(row0, RPT)], aouts[ch].at[c, pl.ds(row0, RPT)])


_sc_agg = pl.kernel(
    _sc_body,
    out_type=[jax.ShapeDtypeStruct((2, NP, FC), jnp.float32) for _ in range(NCH)],
    mesh=plsc.VectorSubcoreMesh(core_axis_name="c", subcore_axis_name="s"),
    scratch_types=[
        pltpu.VMEM((2, EB), jnp.int32),
        pltpu.VMEM((2, EB), jnp.int32),
        pltpu.VMEM((EB, FC), jnp.float32),
        pltpu.VMEM((EB, FC), jnp.float32),
        pltpu.VMEM_SHARED((NP, FC), jnp.float32),
        pltpu.SemaphoreType.DMA,
        pltpu.SemaphoreType.DMA,
        pltpu.SemaphoreType.DMA,
        pltpu.SemaphoreType.DMA,
    ],
    name="sc_edge_agg",
)


def _sc_stats_body(dst_h, ea_h, zer_h, st_h, dst_i, erow, acc2):
    c = lax.axis_index("c")
    s = lax.axis_index("s")
    w = c * 16 + s
    row0 = s * RPT

    pltpu.sync_copy(zer_h, acc2.at[pl.ds(row0, RPT)])
    plsc.subcore_barrier()

    @pl.loop(0, NB)
    def _stats(j):
        pltpu.sync_copy(dst_h.at[pl.ds(w * EPW + j * EB, EB)], dst_i)
        pltpu.sync_copy(ea_h.at[pl.ds(w * EPW + j * EB, EB)], erow)
        pltpu.sync_copy(erow, acc2.at[dst_i], add=True)

    plsc.subcore_barrier()
    pltpu.sync_copy(acc2.at[pl.ds(row0, RPT)], st_h.at[c, pl.ds(row0, RPT)])


_sc_stats = pl.kernel(
    _sc_stats_body,
    out_type=[jax.ShapeDtypeStruct((2, NP, FC), jnp.float32)],
    mesh=plsc.VectorSubcoreMesh(core_axis_name="c", subcore_axis_name="s"),
    scratch_types=[
        pltpu.VMEM((EB,), jnp.int32),
        pltpu.VMEM((EB, FC), jnp.float32),
        pltpu.VMEM_SHARED((NP, FC), jnp.float32),
    ],
    name="sc_edge_stats",
)


def _k1_body(x_ref, wm_ref, ws_ref, bs_ref, *outs):
    xa = x_ref[...]
    pre = jnp.dot(xa, wm_ref[...], preferred_element_type=jnp.float32)
    for ch in range(NCH):
        outs[ch][...] = pre[:, ch * FC:(ch + 1) * FC]
    outs[NCH][...] = jnp.dot(xa, ws_ref[...],
                             preferred_element_type=jnp.float32) + bs_ref[...]


_k1 = pl.pallas_call(
    _k1_body,
    grid=(G,),
    in_specs=[
        pl.BlockSpec((RB, IN), lambda i: (i, 0)),
        pl.BlockSpec((IN, HID), lambda i: (0, 0)),
        pl.BlockSpec((IN, HID), lambda i: (0, 0)),
        pl.BlockSpec((1, HID), lambda i: (0, 0)),
    ],
    out_specs=[pl.BlockSpec((RB, FC), lambda i: (i, 0)) for _ in range(NCH)]
    + [pl.BlockSpec((RB, HID), lambda i: (i, 0))],
    out_shape=[jax.ShapeDtypeStruct((N, FC), jnp.float32) for _ in range(NCH)]
    + [jax.ShapeDtypeStruct((N, HID), jnp.float32)],
)


def _mid_body(residual, *refs):
    a_refs = refs[:NCH]
    st_ref, prev_ref, we_ref, bb_ref, wmn_ref, x_ref = refs[NCH:NCH + 6]
    q_refs = refs[NCH + 6:]
    aggs = []
    for a in a_refs:
        av = a[...]
        aggs.append(av[0] + av[1])
    agg = jnp.concatenate(aggs, axis=1)
    st = st_ref[...]
    sv = st[0] + st[1]
    d = jnp.dot(sv[:, :ED], we_ref[...], preferred_element_type=jnp.float32)
    d = d + sv[:, ED:ED + 1] * bb_ref[...]
    prev = prev_ref[...]
    xl = jnp.maximum(agg + d + prev, 0.0)
    if residual:
        xl = xl + prev
    x_ref[...] = xl
    q = jnp.dot(xl, wmn_ref[...], preferred_element_type=jnp.float32)
    for ch, qr in enumerate(q_refs):
        qr[...] = q[:, ch * FC:(ch + 1) * FC]


def _make_mid(residual):
    return pl.pallas_call(
        functools.partial(_mid_body, residual),
        grid=(G,),
        in_specs=[pl.BlockSpec((2, RB, FC), lambda i: (0, i, 0)) for _ in range(NCH)]
        + [
            pl.BlockSpec((2, RB, FC), lambda i: (0, i, 0)),
            pl.BlockSpec((RB, HID), lambda i: (i, 0)),
            pl.BlockSpec((ED, HID), lambda i: (0, 0)),
            pl.BlockSpec((1, HID), lambda i: (0, 0)),
            pl.BlockSpec((HID, HID), lambda i: (0, 0)),
        ],
        out_specs=[pl.BlockSpec((RB, HID), lambda i: (i, 0))]
        + [pl.BlockSpec((RB, FC), lambda i: (i, 0)) for _ in range(NCH)],
        out_shape=[jax.ShapeDtypeStruct((N, HID), jnp.float32)]
        + [jax.ShapeDtypeStruct((N, FC), jnp.float32) for _ in range(NCH)],
    )


_mid1 = _make_mid(False)
_mid2 = _make_mid(True)


def _e3_body(*refs):
    a_refs = refs[:NCH]
    st_ref, prev_ref, we_ref, bb_ref, b_ref, psum_ref, cnt_ref = refs[NCH:]
    i = pl.program_id(0)
    aggs = []
    for a in a_refs:
        av = a[...]
        aggs.append(av[0] + av[1])
    agg = jnp.concatenate(aggs, axis=1)
    st = st_ref[...]
    sv = st[0] + st[1]
    d = jnp.dot(sv[:, :ED], we_ref[...], preferred_element_type=jnp.float32)
    d = d + sv[:, ED:ED + 1] * bb_ref[...]
    prev = prev_ref[...]
    x3 = jnp.maximum(agg + d + prev, 0.0) + prev
    b = b_ref[0, 0]
    gid = lax.broadcasted_iota(jnp.int32, (NG, RB), 0)
    mask = (gid == b[None, :]).astype(jnp.float32)

    @pl.when(i == 0)
    def _():
        psum_ref[...] = jnp.zeros_like(psum_ref)
        cnt_ref[...] = jnp.zeros_like(cnt_ref)

    psum_ref[...] += jnp.dot(mask, x3, preferred_element_type=jnp.float32)
    cnt_ref[...] += jnp.sum(mask, axis=1, keepdims=True)


_e3 = pl.pallas_call(
    _e3_body,
    grid=(G,),
    in_specs=[pl.BlockSpec((2, RB, FC), lambda i: (0, i, 0)) for _ in range(NCH)]
    + [
        pl.BlockSpec((2, RB, FC), lambda i: (0, i, 0)),
        pl.BlockSpec((RB, HID), lambda i: (i, 0)),
        pl.BlockSpec((ED, HID), lambda i: (0, 0)),
        pl.BlockSpec((1, HID), lambda i: (0, 0)),
        pl.BlockSpec((1, 1, RB), lambda i: (i, 0, 0)),
    ],
    out_specs=[
        pl.BlockSpec((NG, HID), lambda i: (0, 0)),
        pl.BlockSpec((NG, 1), lambda i: (0, 0)),
    ],
    out_shape=[
        jax.ShapeDtypeStruct((NG, HID), jnp.float32),
        jax.ShapeDtypeStruct((NG, 1), jnp.float32),
    ],
)


def _fin_body(psum_ref, cnt_ref, ga_ref, wg_ref, bg_ref, wc_ref, bc_ref,
              wl_ref, bl_ref, out_ref):
    pooled = psum_ref[...] / jnp.maximum(cnt_ref[...], 1.0)
    g = jnp.dot(ga_ref[...], wg_ref[...],
                preferred_element_type=jnp.float32) + bg_ref[...]
    comb = jnp.concatenate([pooled, g], axis=1)
    h = jnp.maximum(jnp.dot(comb, wc_ref[...],
                            preferred_element_type=jnp.float32) + bc_ref[...], 0.0)
    out_ref[...] = jnp.dot(h, wl_ref[...],
                           preferred_element_type=jnp.float32) + bl_ref[...]


_fin = pl.pallas_call(
    _fin_body,
    out_shape=jax.ShapeDtypeStruct((NG, 1), jnp.float32),
)


def kernel(x, edge_index, edge_attr, batch, graph_attr,
           Wm1, bm1, We1, be1, Wm2, bm2, We2, be2,
           Wm3, bm3, We3, be3, Wm4, bm4, We4, be4,
           Ws1, bs1, Wg, bg, Wc, bc, Wl, bl):
    pad_dst = jnp.broadcast_to(
        N + (jnp.arange(PAD, dtype=jnp.int32) % (NP - N - 16)), (NW, PAD))
    src_w = jnp.concatenate(
        [edge_index[0].reshape(NW, EPW0),
         jnp.zeros((NW, PAD), jnp.int32)], axis=1).reshape(NW, NB, EB)
    dst_w = jnp.concatenate(
        [edge_index[1].reshape(NW, EPW0), pad_dst], axis=1).reshape(NW, NB, EB)
    sd = jnp.stack([src_w, dst_w], axis=2)
    dst = dst_w.reshape(NW * EPW)
    ea7 = jnp.concatenate([edge_attr, jnp.ones((E, 1), jnp.float32)], axis=1)
    ea7 = jnp.concatenate(
        [ea7.reshape(NW, EPW0, ED + 1),
         jnp.zeros((NW, PAD, ED + 1), jnp.float32)], axis=1)
    ea128 = jnp.pad(ea7.reshape(NW * EPW, ED + 1),
                    ((0, 0), (0, FC - ED - 1)))

    *p, xself = _k1(x, Wm1, Ws1, bs1.reshape(1, HID))

    zer = jnp.zeros((RPT, FC), jnp.float32)
    (stats,) = _sc_stats(dst, ea128, zer)
    a = _sc_agg(sd, zer, *p)
    x1, *q = _mid1(*a, stats, xself, We1, (bm1 + be1).reshape(1, HID), Wm2)

    a = _sc_agg(sd, zer, *q)
    x2, *q = _mid2(*a, stats, x1, We2, (bm2 + be2).reshape(1, HID), Wm3)

    a = _sc_agg(sd, zer, *q)
    psum, cnt = _e3(*a, stats, x2, We3,
                    (bm3 + be3).reshape(1, HID), batch.reshape(G, 1, RB))

    return _fin(psum, cnt, graph_attr, Wg, bg.reshape(1, HID),
                Wc, bc.reshape(1, HID), Wl, bl.reshape(1, 1))
